# baseline (speedup 1.0000x reference)
import functools
import jax
import jax.numpy as jnp
from jax import lax
from jax.experimental import pallas as pl
from jax.experimental.pallas import tpu as pltpu
from jax.experimental.pallas import tpu_sc as plsc

B, C, N, K, O = 8, 128, 4096, 16, 128
NW = 32
ROWS = B * N
RPW = ROWS // NW
CH = 16
NCH = RPW // CH
NIL = 128
NB = 2048
NB2 = 2048


def _matmul_body(f_ref, w12_ref, b12_ref, wd_ref, bd_ref, t12_ref, d_ref):
    fb = f_ref[0]
    cn = (((0,), (1,)), ((), ()))
    g12 = lax.dot_general(fb, w12_ref[...], cn, preferred_element_type=jnp.float32)
    t12_ref[0] = (g12 + b12_ref[...]).astype(jnp.bfloat16).reshape(NB, 2, O)
    d = lax.dot_general(fb, wd_ref[...], cn, preferred_element_type=jnp.float32)
    d_ref[0] = (d + bd_ref[...]).astype(jnp.bfloat16).reshape(NB // 2, 2, O)


def _sc_body(t12_hbm, idx_hbm, d_hbm, h_hbm,
             idx_v, g0, g1, g2, d0, d1, d2, h0, h1, h2,
             sem_g0, sem_g1, sem_g2, sem_d0, sem_d1, sem_d2,
             sem_h0, sem_h1, sem_h2):
    wid = lax.axis_index("s") * 2 + lax.axis_index("c")
    base = wid * RPW

    pltpu.sync_copy(idx_hbm.at[pl.ds(wid * 2 * NCH, 2 * NCH)], idx_v)

    gbufs = (g0, g1, g2)
    dbufs = (d0, d1, d2)
    hbufs = (h0, h1, h2)
    gsems = (sem_g0, sem_g1, sem_g2)
    dsems = (sem_d0, sem_d1, sem_d2)
    hsems = (sem_h0, sem_h1, sem_h2)
    t32 = t12_hbm.bitcast(jnp.int32)
    g32 = tuple(g.bitcast(jnp.int32) for g in gbufs)
    d32h = d_hbm.bitcast(jnp.int32)
    d32 = tuple(dd.bitcast(jnp.int32) for dd in dbufs)
    h32h = h_hbm.bitcast(jnp.int32)
    h32 = tuple(hh.bitcast(jnp.int32) for hh in hbufs)
    base2 = base // 2
    CH2 = CH // 2

    def g_copy_a(c, bi):
        return pltpu.make_async_copy(t32.at[idx_v.at[2 * c]],
                                     g32[bi].at[pl.ds(0, NIL)], gsems[bi])

    def g_copy_b(c, bi):
        return pltpu.make_async_copy(t32.at[idx_v.at[2 * c + 1]],
                                     g32[bi].at[pl.ds(NIL, NIL)], gsems[bi])

    def g_start(c, bi):
        g_copy_a(c, bi).start()
        g_copy_b(c, bi).start()

    def g_wait(c, bi):
        g_copy_a(c, bi).wait()
        g_copy_b(c, bi).wait()

    def d_copy(c, bi):
        return pltpu.make_async_copy(d32h.at[pl.ds(base2 + c * CH2, CH2)],
                                     d32[bi], dsems[bi])

    def h_copy(c, bi):
        return pltpu.make_async_copy(h32[bi],
                                     h32h.at[pl.ds(base2 + c * CH2, CH2)],
                                     hsems[bi])

    g_start(0, 0)
    d_copy(0, 0).start()
    g_start(1, 1)
    d_copy(1, 1).start()
    g_start(2, 2)
    d_copy(2, 2).start()

    def pair_body(i, carry):
        for bi in range(3):
            c = 3 * i + bi
            g_wait(c, bi)
            d_copy(c, bi).wait()

            @pl.when(c >= 3)
            def _():
                h_copy(c - 3, bi).wait()

            gb = gbufs[bi]
            db = dbufs[bi]
            hb = hbufs[bi]

            def row_body(r2, rcarry):
                dslab = db[r2].astype(jnp.float32)
                hrows = []
                for p in range(2):
                    rb = (2 * r2 + p) * K
                    m = gb[rb]
                    for k_ in range(1, K):
                        m = jnp.maximum(m, gb[rb + k_])
                    mf = m.astype(jnp.float32)
                    hrow = mf[0] + mf[1] + dslab[p]
                    hrows.append(hrow[None])
                h2 = jnp.concatenate(hrows, axis=0).astype(jnp.bfloat16)
                hb[r2] = h2
                return rcarry

            lax.fori_loop(0, CH2, row_body, 0)
            h_copy(c, bi).start()

            nc = c + 3

            @pl.when(nc < NCH)
            def _():
                g_start(nc, bi)
                d_copy(nc, bi).start()
        return carry

    lax.fori_loop(0, (NCH - 1) // 3, pair_body, 0)
    ct = NCH - 1
    g_wait(ct, 0)
    d_copy(ct, 0).wait()
    h_copy(ct - 3, 0).wait()
    gb, db, hb = gbufs[0], dbufs[0], hbufs[0]

    def tail_row(r2, rcarry):
        dslab = db[r2].astype(jnp.float32)
        hrows = []
        for p in range(2):
            rb = (2 * r2 + p) * K
            m = gb[rb]
            for k_ in range(1, K):
                m = jnp.maximum(m, gb[rb + k_])
            mf = m.astype(jnp.float32)
            hrow = mf[0] + mf[1] + dslab[p]
            hrows.append(hrow[None])
        hb[r2] = jnp.concatenate(hrows, axis=0).astype(jnp.bfloat16)
        return rcarry

    lax.fori_loop(0, CH2, tail_row, 0)
    h_copy(ct, 0).start()
    h_copy(NCH - 3, 1).wait()
    h_copy(NCH - 2, 2).wait()
    h_copy(ct, 0).wait()


def _stats_body(h_ref, stats_ref):
    @pl.when((pl.program_id(0) == 0) & (pl.program_id(1) == 0))
    def _():
        stats_ref[...] = jnp.zeros((1, 2 * O), jnp.float32)
    hb = h_ref[0].astype(jnp.float32)
    s = jnp.sum(hb, axis=0)
    sq = jnp.sum(hb * hb, axis=0)
    stats_ref[0, :O] += s
    stats_ref[0, O:] += sq


def _norm_body(h_ref, stats_ref, gamma_ref, beta_ref, out_ref):
    s = stats_ref[...]
    ssum = s[0, :128]
    ssq = s[0, 128:]
    cnt = jnp.float32(ROWS)
    mean = ssum / cnt
    var = ssq / cnt - mean * mean
    scale = gamma_ref[0] * lax.rsqrt(var + 1e-5)
    shift = beta_ref[0] - mean * scale
    hb = h_ref[0].astype(jnp.float32)
    out_ref[0] = hb.T * scale[:, None] + shift[:, None]


def kernel(f, idx, W1, b1, W2, b2, W3, b3, gamma, beta):
    W12 = jnp.concatenate([W1, W2], axis=0)
    b12 = jnp.concatenate([b1, b2])[None, :]
    Wd = W3 - W1
    bd = (b3 - b1)[None, :]

    t12, d = pl.pallas_call(
        _matmul_body,
        grid=(B, N // NB),
        in_specs=[
            pl.BlockSpec((1, C, NB), lambda b, n: (b, 0, n)),
            pl.BlockSpec((2 * O, C), lambda b, n: (0, 0)),
            pl.BlockSpec((1, 2 * O), lambda b, n: (0, 0)),
            pl.BlockSpec((O, C), lambda b, n: (0, 0)),
            pl.BlockSpec((1, O), lambda b, n: (0, 0)),
        ],
        out_specs=[
            pl.BlockSpec((1, NB, 2, O), lambda b, n: (b, n, 0, 0)),
            pl.BlockSpec((1, NB // 2, 2, O), lambda b, n: (b, n, 0, 0)),
        ],
        out_shape=[
            jax.ShapeDtypeStruct((B, N, 2, O), jnp.bfloat16),
            jax.ShapeDtypeStruct((B, N // 2, 2, O), jnp.bfloat16),
        ],
    )(f, W12, b12, Wd, bd)

    t12 = t12.reshape(ROWS, 2, O)
    d2 = d.reshape(ROWS // 2, 2, O)
    idxg = (idx + (jnp.arange(B, dtype=jnp.int32) * N)[:, None, None])
    idxg = idxg.reshape(NW * 2 * NCH, NIL)

    mesh = plsc.VectorSubcoreMesh(core_axis_name="c", subcore_axis_name="s",
                                  num_cores=2, num_subcores=16)
    sc = pl.kernel(
        _sc_body,
        mesh=mesh,
        out_type=jax.ShapeDtypeStruct((ROWS // 2, 2, O), jnp.bfloat16),
        scratch_types=[
            pltpu.VMEM((2 * NCH, NIL), jnp.int32),
            pltpu.VMEM((CH * K, 2, O), jnp.bfloat16),
            pltpu.VMEM((CH * K, 2, O), jnp.bfloat16),
            pltpu.VMEM((CH * K, 2, O), jnp.bfloat16),
            pltpu.VMEM((CH // 2, 2, O), jnp.bfloat16),
            pltpu.VMEM((CH // 2, 2, O), jnp.bfloat16),
            pltpu.VMEM((CH // 2, 2, O), jnp.bfloat16),
            pltpu.VMEM((CH // 2, 2, O), jnp.bfloat16),
            pltpu.VMEM((CH // 2, 2, O), jnp.bfloat16),
            pltpu.VMEM((CH // 2, 2, O), jnp.bfloat16),
            pltpu.SemaphoreType.DMA,
            pltpu.SemaphoreType.DMA,
            pltpu.SemaphoreType.DMA,
            pltpu.SemaphoreType.DMA,
            pltpu.SemaphoreType.DMA,
            pltpu.SemaphoreType.DMA,
            pltpu.SemaphoreType.DMA,
            pltpu.SemaphoreType.DMA,
            pltpu.SemaphoreType.DMA,
        ],
    )
    h = sc(t12, idxg, d2)

    h3 = h.reshape(B, N, O)
    stats = pl.pallas_call(
        _stats_body,
        grid=(B, N // NB2),
        in_specs=[pl.BlockSpec((1, NB2, O), lambda b, n: (b, n, 0))],
        out_specs=pl.BlockSpec((1, 2 * O), lambda b, n: (0, 0)),
        out_shape=jax.ShapeDtypeStruct((1, 2 * O), jnp.float32),
    )(h3)
    out = pl.pallas_call(
        _norm_body,
        grid=(B, N // NB2),
        in_specs=[
            pl.BlockSpec((1, NB2, O), lambda b, n: (b, n, 0)),
            pl.BlockSpec((1, 2 * O), lambda b, n: (0, 0)),
            pl.BlockSpec((1, O), lambda b, n: (0, 0)),
            pl.BlockSpec((1, O), lambda b, n: (0, 0)),
        ],
        out_specs=pl.BlockSpec((1, O, NB2), lambda b, n: (b, 0, n)),
        out_shape=jax.ShapeDtypeStruct((B, O, N), jnp.float32),
    )(h3, stats, gamma[None, :], beta[None, :])
    return out


# speedup vs baseline: 1.4430x; 1.4430x over previous
import functools
import jax
import jax.numpy as jnp
from jax import lax
from jax.experimental import pallas as pl
from jax.experimental.pallas import tpu as pltpu
from jax.experimental.pallas import tpu_sc as plsc

B, C, N, K, O = 8, 128, 4096, 16, 128
NW = 32
ROWS = B * N
BH = B // 2
HROWS = ROWS // 2
RPW = HROWS // NW
CH = 16
NCH = RPW // CH
NIL = 128
NB = 2048
NB2 = 2048


def _matmul_body(f_ref, w12_ref, b12_ref, wd_ref, bd_ref, t12_ref, d_ref):
    fb = f_ref[0]
    cn = (((0,), (1,)), ((), ()))
    g12 = lax.dot_general(fb, w12_ref[...], cn, preferred_element_type=jnp.float32)
    t12_ref[0] = (g12 + b12_ref[...]).astype(jnp.bfloat16).reshape(NB, 2, O)
    d = lax.dot_general(fb, wd_ref[...], cn, preferred_element_type=jnp.float32)
    d_ref[0] = (d + bd_ref[...]).astype(jnp.bfloat16).reshape(NB // 2, 2, O)


def _sc_body(t12_hbm, idx_hbm, d_hbm, h_hbm, stats_hbm,
             idx_v, g0, g1, g2, d0, d1, d2, h0, h1, h2, stats_v,
             sem_g0, sem_g1, sem_g2, sem_d0, sem_d1, sem_d2,
             sem_h0, sem_h1, sem_h2):
    wid = lax.axis_index("s") * 2 + lax.axis_index("c")
    base = wid * RPW

    pltpu.sync_copy(idx_hbm.at[pl.ds(wid * 2 * NCH, 2 * NCH)], idx_v)

    gbufs = (g0, g1, g2)
    dbufs = (d0, d1, d2)
    hbufs = (h0, h1, h2)
    gsems = (sem_g0, sem_g1, sem_g2)
    dsems = (sem_d0, sem_d1, sem_d2)
    hsems = (sem_h0, sem_h1, sem_h2)
    t32 = t12_hbm.bitcast(jnp.int32)
    g32 = tuple(g.bitcast(jnp.int32) for g in gbufs)
    d32h = d_hbm.bitcast(jnp.int32)
    d32 = tuple(dd.bitcast(jnp.int32) for dd in dbufs)
    h32h = h_hbm.bitcast(jnp.int32)
    h32 = tuple(hh.bitcast(jnp.int32) for hh in hbufs)
    base2 = base // 2
    CH2 = CH // 2

    def g_copy_a(c, bi):
        return pltpu.make_async_copy(t32.at[idx_v.at[2 * c]],
                                     g32[bi].at[pl.ds(0, NIL)], gsems[bi])

    def g_copy_b(c, bi):
        return pltpu.make_async_copy(t32.at[idx_v.at[2 * c + 1]],
                                     g32[bi].at[pl.ds(NIL, NIL)], gsems[bi])

    def g_start(c, bi):
        g_copy_a(c, bi).start()
        g_copy_b(c, bi).start()

    def g_wait(c, bi):
        g_copy_a(c, bi).wait()
        g_copy_b(c, bi).wait()

    def d_copy(c, bi):
        return pltpu.make_async_copy(d32h.at[pl.ds(base2 + c * CH2, CH2)],
                                     d32[bi], dsems[bi])

    def h_copy(c, bi):
        return pltpu.make_async_copy(h32[bi],
                                     h32h.at[pl.ds(base2 + c * CH2, CH2)],
                                     hsems[bi])

    stats_v[...] = jnp.zeros((2 * O,), jnp.float32)

    for c0 in range(3):
        g_start(c0, c0)
        d_copy(c0, c0).start()

    def process(c, bi, prefetch):
        g_wait(c, bi)
        d_copy(c, bi).wait()

        @pl.when(c >= 3)
        def _():
            h_copy(c - 3, bi).wait()

        gb = gbufs[bi]
        db = dbufs[bi]
        hb = hbufs[bi]

        def row_body(r2, rcarry):
            dslab = db[r2].astype(jnp.float32)
            hrows = []
            for p in range(2):
                rb = (2 * r2 + p) * K
                m = gb[rb]
                for k_ in range(1, K):
                    m = jnp.maximum(m, gb[rb + k_])
                mf = m.astype(jnp.float32)
                hrow = mf[0] + mf[1] + dslab[p]
                plsc.addupdate(stats_v.at[pl.ds(0, O)], hrow)
                plsc.addupdate(stats_v.at[pl.ds(O, O)], hrow * hrow)
                hrows.append(hrow[None])
            h2 = jnp.concatenate(hrows, axis=0).astype(jnp.bfloat16)
            hb[r2] = h2
            return rcarry

        lax.fori_loop(0, CH2, row_body, 0)
        h_copy(c, bi).start()

        if prefetch:
            nc = c + 3

            @pl.when(nc < NCH)
            def _():
                g_start(nc, bi)
                d_copy(nc, bi).start()

    NMAIN = 3 * (NCH // 3)

    def main_body(i, carry):
        for bi in range(3):
            process(3 * i + bi, bi, True)
        return carry

    lax.fori_loop(0, NCH // 3, main_body, 0)
    for ct in range(NMAIN, NCH):
        process(ct, ct % 3, False)
    for ct in range(NCH - 3, NCH):
        h_copy(ct, ct % 3).wait()
    pltpu.sync_copy(stats_v, stats_hbm.at[wid])


def _norm_body(ha_ref, hb_ref, stats_ref, gamma_ref, beta_ref, out_ref):
    s = stats_ref[...]
    ssum = jnp.sum(s[:, :128], axis=0)
    ssq = jnp.sum(s[:, 128:], axis=0)
    cnt = jnp.float32(ROWS)
    mean = ssum / cnt
    var = ssq / cnt - mean * mean
    scale = gamma_ref[0] * lax.rsqrt(var + 1e-5)
    shift = beta_ref[0] - mean * scale
    first = pl.program_id(0) < BH
    hblk = jnp.where(first, ha_ref[0], hb_ref[0]).astype(jnp.float32)
    out_ref[0] = hblk.T * scale[:, None] + shift[:, None]


def _half(f, idx, W12, b12, Wd, bd, hf):
    t12, d = pl.pallas_call(
        _matmul_body,
        grid=(BH, N // NB),
        in_specs=[
            pl.BlockSpec((1, C, NB), lambda b, n: (b + hf * BH, 0, n)),
            pl.BlockSpec((2 * O, C), lambda b, n: (0, 0)),
            pl.BlockSpec((1, 2 * O), lambda b, n: (0, 0)),
            pl.BlockSpec((O, C), lambda b, n: (0, 0)),
            pl.BlockSpec((1, O), lambda b, n: (0, 0)),
        ],
        out_specs=[
            pl.BlockSpec((1, NB, 2, O), lambda b, n: (b, n, 0, 0)),
            pl.BlockSpec((1, NB // 2, 2, O), lambda b, n: (b, n, 0, 0)),
        ],
        out_shape=[
            jax.ShapeDtypeStruct((BH, N, 2, O), jnp.bfloat16),
            jax.ShapeDtypeStruct((BH, N // 2, 2, O), jnp.bfloat16),
        ],
    )(f, W12, b12, Wd, bd)

    t12 = t12.reshape(HROWS, 2, O)
    d2 = d.reshape(HROWS // 2, 2, O)
    bidx = lax.slice_in_dim(idx, hf * BH, (hf + 1) * BH, axis=0)
    idxg = (bidx + (jnp.arange(BH, dtype=jnp.int32) * N)[:, None, None])
    idxg = idxg.reshape(NW * 2 * NCH, NIL)

    mesh = plsc.VectorSubcoreMesh(core_axis_name="c", subcore_axis_name="s",
                                  num_cores=2, num_subcores=16)
    sc = pl.kernel(
        _sc_body,
        mesh=mesh,
        out_type=[
            jax.ShapeDtypeStruct((HROWS // 2, 2, O), jnp.bfloat16),
            jax.ShapeDtypeStruct((NW, 2 * O), jnp.float32),
        ],
        scratch_types=[
            pltpu.VMEM((2 * NCH, NIL), jnp.int32),
            pltpu.VMEM((CH * K, 2, O), jnp.bfloat16),
            pltpu.VMEM((CH * K, 2, O), jnp.bfloat16),
            pltpu.VMEM((CH * K, 2, O), jnp.bfloat16),
            pltpu.VMEM((CH // 2, 2, O), jnp.bfloat16),
            pltpu.VMEM((CH // 2, 2, O), jnp.bfloat16),
            pltpu.VMEM((CH // 2, 2, O), jnp.bfloat16),
            pltpu.VMEM((CH // 2, 2, O), jnp.bfloat16),
            pltpu.VMEM((CH // 2, 2, O), jnp.bfloat16),
            pltpu.VMEM((CH // 2, 2, O), jnp.bfloat16),
            pltpu.VMEM((2 * O,), jnp.float32),
            pltpu.SemaphoreType.DMA,
            pltpu.SemaphoreType.DMA,
            pltpu.SemaphoreType.DMA,
            pltpu.SemaphoreType.DMA,
            pltpu.SemaphoreType.DMA,
            pltpu.SemaphoreType.DMA,
            pltpu.SemaphoreType.DMA,
            pltpu.SemaphoreType.DMA,
            pltpu.SemaphoreType.DMA,
        ],
    )
    h, stats = sc(t12, idxg, d2)
    return h.reshape(BH, N, O), stats


def kernel(f, idx, W1, b1, W2, b2, W3, b3, gamma, beta):
    W12 = jnp.concatenate([W1, W2], axis=0)
    b12 = jnp.concatenate([b1, b2])[None, :]
    Wd = W3 - W1
    bd = (b3 - b1)[None, :]

    ha, statsa = _half(f, idx, W12, b12, Wd, bd, 0)
    hb, statsb = _half(f, idx, W12, b12, Wd, bd, 1)
    stats = jnp.concatenate([statsa, statsb], axis=0)

    out = pl.pallas_call(
        _norm_body,
        grid=(B, N // NB2),
        in_specs=[
            pl.BlockSpec((1, NB2, O),
                         lambda b, n: (jnp.minimum(b, BH - 1), n, 0)),
            pl.BlockSpec((1, NB2, O),
                         lambda b, n: (jnp.maximum(b, BH) - BH, n, 0)),
            pl.BlockSpec((2 * NW, 2 * O), lambda b, n: (0, 0)),
            pl.BlockSpec((1, O), lambda b, n: (0, 0)),
            pl.BlockSpec((1, O), lambda b, n: (0, 0)),
        ],
        out_specs=pl.BlockSpec((1, O, NB2), lambda b, n: (b, 0, n)),
        out_shape=jax.ShapeDtypeStruct((B, O, N), jnp.float32),
    )(ha, hb, stats, gamma[None, :], beta[None, :])
    return out


# speedup vs baseline: 1.4810x; 1.0263x over previous
import functools
import jax
import jax.numpy as jnp
from jax import lax
from jax.experimental import pallas as pl
from jax.experimental.pallas import tpu as pltpu
from jax.experimental.pallas import tpu_sc as plsc

B, C, N, K, O = 8, 128, 4096, 16, 128
NW = 32
ROWS = B * N
RPW = ROWS // NW
CH = 16
NCH = RPW // CH
NIL = 128
NB = 2048
NB2 = 2048


def _matmul_body(f_ref, w12_ref, b12_ref, wd_ref, bd_ref, t12_ref, d_ref):
    fb = f_ref[0]
    cn = (((0,), (1,)), ((), ()))
    g12 = lax.dot_general(fb, w12_ref[...], cn, preferred_element_type=jnp.float32)
    t12_ref[0] = (g12 + b12_ref[...]).astype(jnp.bfloat16).reshape(NB, 2, O)
    d = lax.dot_general(fb, wd_ref[...], cn, preferred_element_type=jnp.float32)
    d_ref[0] = (d + bd_ref[...]).astype(jnp.bfloat16).reshape(NB // 2, 2, O)


def _sc_body(t12_hbm, idx_hbm, d_hbm, h_hbm, stats_hbm,
             idx_v, g0, g1, g2, d0, d1, d2, h0, h1, h2, stats_v,
             sem_g0, sem_g1, sem_g2, sem_d0, sem_d1, sem_d2,
             sem_h0, sem_h1, sem_h2):
    wid = lax.axis_index("s") * 2 + lax.axis_index("c")
    base = wid * RPW

    pltpu.sync_copy(idx_hbm.at[pl.ds(wid * 2 * NCH, 2 * NCH)], idx_v)

    gbufs = (g0, g1, g2)
    dbufs = (d0, d1, d2)
    hbufs = (h0, h1, h2)
    gsems = (sem_g0, sem_g1, sem_g2)
    dsems = (sem_d0, sem_d1, sem_d2)
    hsems = (sem_h0, sem_h1, sem_h2)
    t32 = t12_hbm.bitcast(jnp.int32)
    g32 = tuple(g.bitcast(jnp.int32) for g in gbufs)
    d32h = d_hbm.bitcast(jnp.int32)
    d32 = tuple(dd.bitcast(jnp.int32) for dd in dbufs)
    h32h = h_hbm.bitcast(jnp.int32)
    h32 = tuple(hh.bitcast(jnp.int32) for hh in hbufs)
    base2 = base // 2
    CH2 = CH // 2

    def g_copy_a(c, bi):
        return pltpu.make_async_copy(t32.at[idx_v.at[2 * c]],
                                     g32[bi].at[pl.ds(0, NIL)], gsems[bi])

    def g_copy_b(c, bi):
        return pltpu.make_async_copy(t32.at[idx_v.at[2 * c + 1]],
                                     g32[bi].at[pl.ds(NIL, NIL)], gsems[bi])

    def g_start(c, bi):
        g_copy_a(c, bi).start()
        g_copy_b(c, bi).start()

    def g_wait(c, bi):
        g_copy_a(c, bi).wait()
        g_copy_b(c, bi).wait()

    def d_copy(c, bi):
        return pltpu.make_async_copy(d32h.at[pl.ds(base2 + c * CH2, CH2)],
                                     d32[bi], dsems[bi])

    def h_copy(c, bi):
        return pltpu.make_async_copy(h32[bi],
                                     h32h.at[pl.ds(base2 + c * CH2, CH2)],
                                     hsems[bi])

    stats_v[...] = jnp.zeros((2 * O,), jnp.float32)

    g_start(0, 0)
    d_copy(0, 0).start()
    g_start(1, 1)
    d_copy(1, 1).start()
    g_start(2, 2)
    d_copy(2, 2).start()

    def pair_body(i, carry):
        for bi in range(3):
            c = 3 * i + bi
            g_wait(c, bi)
            d_copy(c, bi).wait()

            @pl.when(c >= 3)
            def _():
                h_copy(c - 3, bi).wait()

            gb = gbufs[bi]
            db = dbufs[bi]
            hb = hbufs[bi]

            def row_body(r2, rcarry):
                dslab = db[r2].astype(jnp.float32)
                hrows = []
                for p in range(2):
                    rb = (2 * r2 + p) * K
                    m = gb[rb]
                    for k_ in range(1, K):
                        m = jnp.maximum(m, gb[rb + k_])
                    mf = m.astype(jnp.float32)
                    hrow = mf[0] + mf[1] + dslab[p]
                    plsc.addupdate(stats_v.at[pl.ds(0, O)], hrow)
                    plsc.addupdate(stats_v.at[pl.ds(O, O)], hrow * hrow)
                    hrows.append(hrow[None])
                h2 = jnp.concatenate(hrows, axis=0).astype(jnp.bfloat16)
                hb[r2] = h2
                return rcarry

            lax.fori_loop(0, CH2, row_body, 0)
            h_copy(c, bi).start()

            nc = c + 3

            @pl.when(nc < NCH)
            def _():
                g_start(nc, bi)
                d_copy(nc, bi).start()
        return carry

    lax.fori_loop(0, (NCH - 1) // 3, pair_body, 0)
    ct = NCH - 1
    g_wait(ct, 0)
    d_copy(ct, 0).wait()
    h_copy(ct - 3, 0).wait()
    gb, db, hb = gbufs[0], dbufs[0], hbufs[0]

    def tail_row(r2, rcarry):
        dslab = db[r2].astype(jnp.float32)
        hrows = []
        for p in range(2):
            rb = (2 * r2 + p) * K
            m = gb[rb]
            for k_ in range(1, K):
                m = jnp.maximum(m, gb[rb + k_])
            mf = m.astype(jnp.float32)
            hrow = mf[0] + mf[1] + dslab[p]
            plsc.addupdate(stats_v.at[pl.ds(0, O)], hrow)
            plsc.addupdate(stats_v.at[pl.ds(O, O)], hrow * hrow)
            hrows.append(hrow[None])
        hb[r2] = jnp.concatenate(hrows, axis=0).astype(jnp.bfloat16)
        return rcarry

    lax.fori_loop(0, CH2, tail_row, 0)
    h_copy(ct, 0).start()
    h_copy(NCH - 3, 1).wait()
    h_copy(NCH - 2, 2).wait()
    h_copy(ct, 0).wait()
    pltpu.sync_copy(stats_v, stats_hbm.at[wid])


def _norm_body(h_ref, stats_ref, gamma_ref, beta_ref, out_ref):
    s = stats_ref[...]
    ssum = jnp.sum(s[:, :128], axis=0)
    ssq = jnp.sum(s[:, 128:], axis=0)
    cnt = jnp.float32(ROWS)
    mean = ssum / cnt
    var = ssq / cnt - mean * mean
    scale = gamma_ref[0] * lax.rsqrt(var + 1e-5)
    shift = beta_ref[0] - mean * scale
    hb = h_ref[0].astype(jnp.float32)
    out_ref[0] = hb.T * scale[:, None] + shift[:, None]


def kernel(f, idx, W1, b1, W2, b2, W3, b3, gamma, beta):
    W12 = jnp.concatenate([W1, W2], axis=0)
    b12 = jnp.concatenate([b1, b2])[None, :]
    Wd = W3 - W1
    bd = (b3 - b1)[None, :]

    t12, d = pl.pallas_call(
        _matmul_body,
        grid=(B, N // NB),
        in_specs=[
            pl.BlockSpec((1, C, NB), lambda b, n: (b, 0, n)),
            pl.BlockSpec((2 * O, C), lambda b, n: (0, 0)),
            pl.BlockSpec((1, 2 * O), lambda b, n: (0, 0)),
            pl.BlockSpec((O, C), lambda b, n: (0, 0)),
            pl.BlockSpec((1, O), lambda b, n: (0, 0)),
        ],
        out_specs=[
            pl.BlockSpec((1, NB, 2, O), lambda b, n: (b, n, 0, 0)),
            pl.BlockSpec((1, NB // 2, 2, O), lambda b, n: (b, n, 0, 0)),
        ],
        out_shape=[
            jax.ShapeDtypeStruct((B, N, 2, O), jnp.bfloat16),
            jax.ShapeDtypeStruct((B, N // 2, 2, O), jnp.bfloat16),
        ],
    )(f, W12, b12, Wd, bd)

    t12 = t12.reshape(ROWS, 2, O)
    d2 = d.reshape(ROWS // 2, 2, O)
    idxg = (idx + (jnp.arange(B, dtype=jnp.int32) * N)[:, None, None])
    idxg = idxg.reshape(NW * 2 * NCH, NIL)

    mesh = plsc.VectorSubcoreMesh(core_axis_name="c", subcore_axis_name="s",
                                  num_cores=2, num_subcores=16)
    sc = pl.kernel(
        _sc_body,
        mesh=mesh,
        out_type=[
            jax.ShapeDtypeStruct((ROWS // 2, 2, O), jnp.bfloat16),
            jax.ShapeDtypeStruct((NW, 2 * O), jnp.float32),
        ],
        scratch_types=[
            pltpu.VMEM((2 * NCH, NIL), jnp.int32),
            pltpu.VMEM((CH * K, 2, O), jnp.bfloat16),
            pltpu.VMEM((CH * K, 2, O), jnp.bfloat16),
            pltpu.VMEM((CH * K, 2, O), jnp.bfloat16),
            pltpu.VMEM((CH // 2, 2, O), jnp.bfloat16),
            pltpu.VMEM((CH // 2, 2, O), jnp.bfloat16),
            pltpu.VMEM((CH // 2, 2, O), jnp.bfloat16),
            pltpu.VMEM((CH // 2, 2, O), jnp.bfloat16),
            pltpu.VMEM((CH // 2, 2, O), jnp.bfloat16),
            pltpu.VMEM((CH // 2, 2, O), jnp.bfloat16),
            pltpu.VMEM((2 * O,), jnp.float32),
            pltpu.SemaphoreType.DMA,
            pltpu.SemaphoreType.DMA,
            pltpu.SemaphoreType.DMA,
            pltpu.SemaphoreType.DMA,
            pltpu.SemaphoreType.DMA,
            pltpu.SemaphoreType.DMA,
            pltpu.SemaphoreType.DMA,
            pltpu.SemaphoreType.DMA,
            pltpu.SemaphoreType.DMA,
        ],
    )
    h, stats = sc(t12, idxg, d2)

    h3 = h.reshape(B, N, O)
    out = pl.pallas_call(
        _norm_body,
        grid=(B, N // NB2),
        in_specs=[
            pl.BlockSpec((1, NB2, O), lambda b, n: (b, n, 0)),
            pl.BlockSpec((NW, 2 * O), lambda b, n: (0, 0)),
            pl.BlockSpec((1, O), lambda b, n: (0, 0)),
            pl.BlockSpec((1, O), lambda b, n: (0, 0)),
        ],
        out_specs=pl.BlockSpec((1, O, NB2), lambda b, n: (b, 0, n)),
        out_shape=jax.ShapeDtypeStruct((B, O, N), jnp.float32),
    )(h3, stats, gamma[None, :], beta[None, :])
    return out
